# baseline (speedup 1.0000x reference)
import jax
import jax.numpy as jnp
from jax.experimental import pallas as pl
from jax.experimental.pallas import tpu as pltpu

E = 320000
N = 10000
E_TILE = 4000
N_TILE = 2000


def _edge_body(r_ref, s_ref, e_ref, g_ref, w1_ref, b1_ref, w2_ref, b2_ref,
               gamma_ref, beta_ref, o_ref):
    bf16 = jnp.bfloat16
    w1 = w1_ref[...].astype(bf16)
    h = jnp.dot(r_ref[...].astype(bf16), w1[0:128],
                preferred_element_type=jnp.float32)
    h = h + jnp.dot(s_ref[...].astype(bf16), w1[128:256],
                    preferred_element_type=jnp.float32)
    h = h + jnp.dot(e_ref[...].astype(bf16), w1[256:272],
                    preferred_element_type=jnp.float32)
    h = h + jnp.dot(g_ref[...].astype(bf16), w1[272:400],
                    preferred_element_type=jnp.float32)
    h = jnp.maximum(h + b1_ref[...], 0.0)
    h = jnp.dot(h.astype(bf16), w2_ref[...].astype(bf16),
                preferred_element_type=jnp.float32) + b2_ref[...]
    h = jnp.maximum(h, 0.0)
    mu = jnp.mean(h, axis=-1, keepdims=True)
    var = jnp.mean((h - mu) * (h - mu), axis=-1, keepdims=True)
    o_ref[...] = (h - mu) * jax.lax.rsqrt(var + 1e-5) * gamma_ref[...] + beta_ref[...]


def _node_body(n_ref, g_ref, r_ref, s_ref, w1_ref, b1_ref, w2_ref, b2_ref,
               gamma_ref, beta_ref, o_ref):
    bf16 = jnp.bfloat16
    w1 = w1_ref[...].astype(bf16)
    h = jnp.dot(n_ref[...].astype(bf16), w1[0:128],
                preferred_element_type=jnp.float32)
    h = h + jnp.dot(g_ref[...].astype(bf16), w1[128:256],
                    preferred_element_type=jnp.float32)
    h = h + jnp.dot(r_ref[...].astype(bf16), w1[256:384],
                    preferred_element_type=jnp.float32)
    h = h + jnp.dot(s_ref[...].astype(bf16), w1[384:512],
                    preferred_element_type=jnp.float32)
    h = jnp.maximum(h + b1_ref[...], 0.0)
    h = jnp.dot(h.astype(bf16), w2_ref[...].astype(bf16),
                preferred_element_type=jnp.float32) + b2_ref[...]
    h = jnp.maximum(h, 0.0)
    mu = jnp.mean(h, axis=-1, keepdims=True)
    var = jnp.mean((h - mu) * (h - mu), axis=-1, keepdims=True)
    o_ref[...] = (h - mu) * jax.lax.rsqrt(var + 1e-5) * gamma_ref[...] + beta_ref[...]


def _global_body(n_ref, e_ref, g_ref, w1_ref, b1_ref, w2_ref, b2_ref, o_ref):
    w1 = w1_ref[...]
    h = jnp.dot(n_ref[...], w1[0:128], preferred_element_type=jnp.float32)
    h = h + jnp.dot(e_ref[...], w1[128:256], preferred_element_type=jnp.float32)
    h = h + jnp.dot(g_ref[...], w1[256:384], preferred_element_type=jnp.float32)
    h = jnp.maximum(h + b1_ref[...], 0.0)
    h = jnp.dot(h, w2_ref[...], preferred_element_type=jnp.float32) + b2_ref[...]
    o_ref[...] = jnp.maximum(h, 0.0)


def _row_spec(tile, width):
    return pl.BlockSpec((tile, width), lambda i: (i, 0))


def _full_spec(shape):
    return pl.BlockSpec(shape, lambda i: tuple(0 for _ in shape))


def kernel(edge_attr, node_attr, global_attr, receiver_attr, sender_attr,
           global_attr_to_edge, global_attr_to_nodes, receiver_attr_to_nodes,
           sender_attr_to_node, node_attr_to_global, edge_attr_to_global,
           eW1, eb1, eW2, eb2, eg, ebt,
           nW1, nb1, nW2, nb2, ng, nbt,
           gW1, gb1, gW2, gb2):
    f32 = jnp.float32

    eb1r = eb1.reshape(1, -1)
    eb2r = eb2.reshape(1, -1)
    egr = eg.reshape(1, -1)
    ebtr = ebt.reshape(1, -1)
    edge_out = pl.pallas_call(
        _edge_body,
        grid=(E // E_TILE,),
        in_specs=[
            _row_spec(E_TILE, 128),
            _row_spec(E_TILE, 128),
            _row_spec(E_TILE, 16),
            _row_spec(E_TILE, 128),
            _full_spec((400, 128)),
            _full_spec((1, 128)),
            _full_spec((128, 128)),
            _full_spec((1, 128)),
            _full_spec((1, 128)),
            _full_spec((1, 128)),
        ],
        out_specs=_row_spec(E_TILE, 128),
        out_shape=jax.ShapeDtypeStruct((E, 128), f32),
        compiler_params=pltpu.CompilerParams(
            dimension_semantics=("parallel",)),
    )(receiver_attr, sender_attr, edge_attr, global_attr_to_edge,
      eW1, eb1r, eW2, eb2r, egr, ebtr)

    nb1r = nb1.reshape(1, -1)
    nb2r = nb2.reshape(1, -1)
    ngr = ng.reshape(1, -1)
    nbtr = nbt.reshape(1, -1)
    node_out = pl.pallas_call(
        _node_body,
        grid=(N // N_TILE,),
        in_specs=[
            _row_spec(N_TILE, 128),
            _row_spec(N_TILE, 128),
            _row_spec(N_TILE, 128),
            _row_spec(N_TILE, 128),
            _full_spec((512, 128)),
            _full_spec((1, 128)),
            _full_spec((128, 128)),
            _full_spec((1, 128)),
            _full_spec((1, 128)),
            _full_spec((1, 128)),
        ],
        out_specs=_row_spec(N_TILE, 128),
        out_shape=jax.ShapeDtypeStruct((N, 128), f32),
        compiler_params=pltpu.CompilerParams(
            dimension_semantics=("parallel",)),
    )(node_attr, global_attr_to_nodes, receiver_attr_to_nodes,
      sender_attr_to_node, nW1, nb1r, nW2, nb2r, ngr, nbtr)

    gb1r = gb1.reshape(1, -1)
    gb2r = gb2.reshape(1, -1)
    global_out = pl.pallas_call(
        _global_body,
        out_shape=jax.ShapeDtypeStruct((1, 128), f32),
    )(node_attr_to_global, edge_attr_to_global, global_attr,
      gW1, gb1r, gW2, gb2r)

    return (edge_out, node_out, global_out)


# speedup vs baseline: 1.0244x; 1.0244x over previous
import jax
import jax.numpy as jnp
from jax.experimental import pallas as pl
from jax.experimental.pallas import tpu as pltpu

E = 320000
N = 10000
E_TILE = 8000
N_TILE = 2000


def _edge_body(r_ref, s_ref, e_ref, g_ref, w1_ref, b1_ref, w2_ref, b2_ref,
               gamma_ref, beta_ref, o_ref):
    bf16 = jnp.bfloat16
    w1 = w1_ref[...].astype(bf16)
    h = jnp.dot(r_ref[...].astype(bf16), w1[0:128],
                preferred_element_type=jnp.float32)
    h = h + jnp.dot(s_ref[...].astype(bf16), w1[128:256],
                    preferred_element_type=jnp.float32)
    h = h + jnp.dot(e_ref[...].astype(bf16), w1[256:272],
                    preferred_element_type=jnp.float32)
    h = h + jnp.dot(g_ref[...].astype(bf16), w1[272:400],
                    preferred_element_type=jnp.float32)
    h = jnp.maximum(h + b1_ref[...], 0.0)
    h = jnp.dot(h.astype(bf16), w2_ref[...].astype(bf16),
                preferred_element_type=jnp.float32) + b2_ref[...]
    h = jnp.maximum(h, 0.0)
    mu = jnp.mean(h, axis=-1, keepdims=True)
    var = jnp.mean((h - mu) * (h - mu), axis=-1, keepdims=True)
    o_ref[...] = (h - mu) * jax.lax.rsqrt(var + 1e-5) * gamma_ref[...] + beta_ref[...]


def _node_body(n_ref, g_ref, r_ref, s_ref, w1_ref, b1_ref, w2_ref, b2_ref,
               gamma_ref, beta_ref, o_ref):
    bf16 = jnp.bfloat16
    w1 = w1_ref[...].astype(bf16)
    h = jnp.dot(n_ref[...].astype(bf16), w1[0:128],
                preferred_element_type=jnp.float32)
    h = h + jnp.dot(g_ref[...].astype(bf16), w1[128:256],
                    preferred_element_type=jnp.float32)
    h = h + jnp.dot(r_ref[...].astype(bf16), w1[256:384],
                    preferred_element_type=jnp.float32)
    h = h + jnp.dot(s_ref[...].astype(bf16), w1[384:512],
                    preferred_element_type=jnp.float32)
    h = jnp.maximum(h + b1_ref[...], 0.0)
    h = jnp.dot(h.astype(bf16), w2_ref[...].astype(bf16),
                preferred_element_type=jnp.float32) + b2_ref[...]
    h = jnp.maximum(h, 0.0)
    mu = jnp.mean(h, axis=-1, keepdims=True)
    var = jnp.mean((h - mu) * (h - mu), axis=-1, keepdims=True)
    o_ref[...] = (h - mu) * jax.lax.rsqrt(var + 1e-5) * gamma_ref[...] + beta_ref[...]


def _global_body(n_ref, e_ref, g_ref, w1_ref, b1_ref, w2_ref, b2_ref, o_ref):
    w1 = w1_ref[...]
    h = jnp.dot(n_ref[...], w1[0:128], preferred_element_type=jnp.float32)
    h = h + jnp.dot(e_ref[...], w1[128:256], preferred_element_type=jnp.float32)
    h = h + jnp.dot(g_ref[...], w1[256:384], preferred_element_type=jnp.float32)
    h = jnp.maximum(h + b1_ref[...], 0.0)
    h = jnp.dot(h, w2_ref[...], preferred_element_type=jnp.float32) + b2_ref[...]
    o_ref[...] = jnp.maximum(h, 0.0)


def _row_spec(tile, width):
    return pl.BlockSpec((tile, width), lambda i: (i, 0))


def _full_spec(shape):
    return pl.BlockSpec(shape, lambda i: tuple(0 for _ in shape))


def kernel(edge_attr, node_attr, global_attr, receiver_attr, sender_attr,
           global_attr_to_edge, global_attr_to_nodes, receiver_attr_to_nodes,
           sender_attr_to_node, node_attr_to_global, edge_attr_to_global,
           eW1, eb1, eW2, eb2, eg, ebt,
           nW1, nb1, nW2, nb2, ng, nbt,
           gW1, gb1, gW2, gb2):
    f32 = jnp.float32

    eb1r = eb1.reshape(1, -1)
    eb2r = eb2.reshape(1, -1)
    egr = eg.reshape(1, -1)
    ebtr = ebt.reshape(1, -1)
    edge_out = pl.pallas_call(
        _edge_body,
        grid=(E // E_TILE,),
        in_specs=[
            _row_spec(E_TILE, 128),
            _row_spec(E_TILE, 128),
            _row_spec(E_TILE, 16),
            _row_spec(E_TILE, 128),
            _full_spec((400, 128)),
            _full_spec((1, 128)),
            _full_spec((128, 128)),
            _full_spec((1, 128)),
            _full_spec((1, 128)),
            _full_spec((1, 128)),
        ],
        out_specs=_row_spec(E_TILE, 128),
        out_shape=jax.ShapeDtypeStruct((E, 128), f32),
        compiler_params=pltpu.CompilerParams(
            dimension_semantics=("parallel",)),
    )(receiver_attr, sender_attr, edge_attr, global_attr_to_edge,
      eW1, eb1r, eW2, eb2r, egr, ebtr)

    nb1r = nb1.reshape(1, -1)
    nb2r = nb2.reshape(1, -1)
    ngr = ng.reshape(1, -1)
    nbtr = nbt.reshape(1, -1)
    node_out = pl.pallas_call(
        _node_body,
        grid=(N // N_TILE,),
        in_specs=[
            _row_spec(N_TILE, 128),
            _row_spec(N_TILE, 128),
            _row_spec(N_TILE, 128),
            _row_spec(N_TILE, 128),
            _full_spec((512, 128)),
            _full_spec((1, 128)),
            _full_spec((128, 128)),
            _full_spec((1, 128)),
            _full_spec((1, 128)),
            _full_spec((1, 128)),
        ],
        out_specs=_row_spec(N_TILE, 128),
        out_shape=jax.ShapeDtypeStruct((N, 128), f32),
        compiler_params=pltpu.CompilerParams(
            dimension_semantics=("parallel",)),
    )(node_attr, global_attr_to_nodes, receiver_attr_to_nodes,
      sender_attr_to_node, nW1, nb1r, nW2, nb2r, ngr, nbtr)

    gb1r = gb1.reshape(1, -1)
    gb2r = gb2.reshape(1, -1)
    global_out = pl.pallas_call(
        _global_body,
        out_shape=jax.ShapeDtypeStruct((1, 128), f32),
    )(node_attr_to_global, edge_attr_to_global, global_attr,
      gW1, gb1r, gW2, gb2r)

    return (edge_out, node_out, global_out)
